# baseline (speedup 1.0000x reference)
import functools

import jax
import jax.numpy as jnp
from jax import lax
from jax.experimental import pallas as pl
from jax.experimental.pallas import tpu as pltpu
from jax.experimental.pallas import tpu_sc as plsc

N = 10000
H = 128
E = 160000
STEPS = 4

NW = 32
EW = 2 * E // NW
KG = 80
ES = E // 16
KS = 80
NSR = 624
NTAIL = N - 16 * NSR
BN = 1000
BE = 2000

_SC_MESH = dict(core_axis_name="c", subcore_axis_name="s")


NCH = EW // KG
NPAIR = (NCH - 1) // 2


def _sc_gather_add(tab_a, tab_b, src, dst):
    @functools.partial(
        pl.kernel,
        mesh=plsc.VectorSubcoreMesh(**_SC_MESH),
        out_type=jax.ShapeDtypeStruct((2 * E, H), jnp.float32),
        scratch_types=[
            pltpu.VMEM((NCH, KG), jnp.int32),
            pltpu.VMEM((NCH, KG), jnp.int32),
            pltpu.VMEM((KG, H), jnp.float32),
            pltpu.VMEM((KG, H), jnp.float32),
            pltpu.VMEM((KG, H), jnp.float32),
            pltpu.VMEM((KG, H), jnp.float32),
            pltpu.SemaphoreType.DMA,
            pltpu.SemaphoreType.DMA,
            pltpu.SemaphoreType.DMA,
            pltpu.SemaphoreType.DMA,
        ],
    )
    def k(ta, tb, s_idx, d_idx, z_out, si, di, ba0, bb0, ba1, bb1,
          g0, g1, w0, w1):
        wid = lax.axis_index("s") * 2 + lax.axis_index("c")
        base = pl.multiple_of(wid * EW, 8)
        pltpu.sync_copy(s_idx.at[wid], si)
        pltpu.sync_copy(d_idx.at[wid], di)

        def gdesc(i, ba, bb, sem):
            return (pltpu.make_async_copy(ta.at[si.at[i]], ba, sem),
                    pltpu.make_async_copy(tb.at[di.at[i]], bb, sem))

        def wdesc(i, ba, sem):
            off = pl.multiple_of(base + i * KG, 8)
            return pltpu.make_async_copy(ba, z_out.at[pl.ds(off, KG)], sem)

        def fire(i, ba, bb, sem):
            for d in gdesc(i, ba, bb, sem):
                d.start()

        def wait_gather(i, ba, bb, sem):
            for d in gdesc(i, ba, bb, sem):
                d.wait()

        def add_pair(ba, bb):
            for r in range(KG):
                for c in range(H // 16):
                    sl = pl.ds(c * 16, 16)
                    plsc.addupdate(ba.at[r, sl], bb[r, sl])

        fire(0, ba0, bb0, g0)
        fire(1, ba1, bb1, g1)

        def body(g, carry):
            i0 = 2 * g
            wait_gather(i0, ba0, bb0, g0)
            add_pair(ba0, bb0)
            wdesc(i0, ba0, w0).start()
            wait_gather(i0 + 1, ba1, bb1, g1)
            add_pair(ba1, bb1)
            wdesc(i0 + 1, ba1, w1).start()
            wdesc(i0, ba0, w0).wait()
            fire(i0 + 2, ba0, bb0, g0)

            @pl.when(g < NPAIR - 1)
            def _refill():
                wdesc(i0 + 1, ba1, w1).wait()
                fire(i0 + 3, ba1, bb1, g1)

            return carry

        lax.fori_loop(0, NPAIR, body, 0)
        wait_gather(NCH - 1, ba0, bb0, g0)
        add_pair(ba0, bb0)
        wdesc(NCH - 1, ba0, w0).start()
        wdesc(NCH - 1, ba0, w0).wait()
        wdesc(NCH - 2, ba1, w1).wait()

    return k(tab_a, tab_b, src, dst)


def _sc_scatter_add(m_all, dst2, zeros_nh):
    NCS = ES // KS

    @functools.partial(
        pl.kernel,
        mesh=plsc.VectorSubcoreMesh(**_SC_MESH),
        out_type=jax.ShapeDtypeStruct((2, N, H), jnp.float32),
        scratch_types=[
            pltpu.VMEM((NCS, KS), jnp.int32),
            pltpu.VMEM((KS, H), jnp.float32),
            pltpu.VMEM((KS, H), jnp.float32),
            pltpu.VMEM_SHARED((N, H), jnp.float32),
            pltpu.SemaphoreType.DMA,
            pltpu.SemaphoreType.DMA,
        ],
    )
    def k(m_hbm, d_idx, z_hbm, agg_out, idx2, m0, m1, agg_sh, r0s, r1s):
        c = lax.axis_index("c")
        s = lax.axis_index("s")
        row0 = pl.multiple_of(s * NSR, 8)

        def rdesc(i, buf, sem):
            off = pl.multiple_of(s * ES + i * KS, 8)
            return pltpu.make_async_copy(m_hbm.at[c, pl.ds(off, KS)],
                                         buf, sem)

        pltpu.sync_copy(d_idx.at[c, s], idx2)
        rdesc(0, m0, r0s).start()
        rdesc(1, m1, r1s).start()
        pltpu.sync_copy(z_hbm.at[pl.ds(row0, NSR)],
                        agg_sh.at[pl.ds(row0, NSR)])

        @pl.when(s == 15)
        def _zero_tail():
            pltpu.sync_copy(z_hbm.at[pl.ds(16 * NSR, NTAIL)],
                            agg_sh.at[pl.ds(16 * NSR, NTAIL)])

        plsc.subcore_barrier()

        def body(g, carry):
            i0 = 2 * g
            rdesc(i0, m0, r0s).wait()
            pltpu.sync_copy(m0, agg_sh.at[idx2.at[i0]], add=True)
            rdesc(i0 + 2, m0, r0s).start()
            rdesc(i0 + 1, m1, r1s).wait()
            pltpu.sync_copy(m1, agg_sh.at[idx2.at[i0 + 1]], add=True)

            @pl.when(g < NPAIR - 1)
            def _refill():
                rdesc(i0 + 3, m1, r1s).start()

            return carry

        lax.fori_loop(0, (NCS - 1) // 2, body, 0)
        rdesc(NCS - 1, m0, r0s).wait()
        pltpu.sync_copy(m0, agg_sh.at[idx2.at[NCS - 1]], add=True)
        plsc.subcore_barrier()
        pltpu.sync_copy(agg_sh.at[pl.ds(row0, NSR)],
                        agg_out.at[c, pl.ds(row0, NSR)])

        @pl.when(s == 15)
        def _write_tail():
            pltpu.sync_copy(agg_sh.at[pl.ds(16 * NSR, NTAIL)],
                            agg_out.at[c, pl.ds(16 * NSR, NTAIL)])

    return k(m_all, dst2, zeros_nh)


def _tc_init(cq2, emb, w1s, w1d, b1):
    def body(q_ref, e_ref, ws_ref, wd_ref, b1_ref, x_ref, a_ref, bt_ref):
        q = q_ref[...]
        e = e_ref[...]
        x = jnp.where(q == 0, e[0:1, :], jnp.where(q == 1, e[1:2, :], e[2:3, :]))
        x_ref[...] = x
        for t in range(2):
            a_ref[t] = jnp.dot(x, ws_ref[t], preferred_element_type=jnp.float32)
            bt_ref[t] = (jnp.dot(x, wd_ref[t],
                                 preferred_element_type=jnp.float32)
                         + b1_ref[t])

    return pl.pallas_call(
        body,
        grid=(N // BN,),
        in_specs=[
            pl.BlockSpec((BN, 1), lambda i: (i, 0)),
            pl.BlockSpec((3, H), lambda i: (0, 0)),
            pl.BlockSpec((2, H, H), lambda i: (0, 0, 0)),
            pl.BlockSpec((2, H, H), lambda i: (0, 0, 0)),
            pl.BlockSpec((2, 1, H), lambda i: (0, 0, 0)),
        ],
        out_specs=[
            pl.BlockSpec((BN, H), lambda i: (i, 0)),
            pl.BlockSpec((2, BN, H), lambda i: (0, i, 0)),
            pl.BlockSpec((2, BN, H), lambda i: (0, i, 0)),
        ],
        out_shape=[
            jax.ShapeDtypeStruct((N, H), jnp.float32),
            jax.ShapeDtypeStruct((2, N, H), jnp.float32),
            jax.ShapeDtypeStruct((2, N, H), jnp.float32),
        ],
    )(cq2, emb, w1s, w1d, b1)


def _tc_mlp(z_all, w2, b2, w3, b3, w4, b4):
    def body(z_ref, w2r, b2r, w3r, b3r, w4r, b4r, m_ref):
        t = jnp.maximum(z_ref[0], 0.0).astype(jnp.bfloat16)
        t = jnp.maximum(
            jnp.dot(t, w2r[0], preferred_element_type=jnp.float32) + b2r[0],
            0.0).astype(jnp.bfloat16)
        t = jnp.maximum(
            jnp.dot(t, w3r[0], preferred_element_type=jnp.float32) + b3r[0],
            0.0).astype(jnp.bfloat16)
        m_ref[0] = jnp.dot(t, w4r[0], preferred_element_type=jnp.float32) + b4r[0]

    wspec = pl.BlockSpec((1, H, H), lambda t, i: (t, 0, 0))
    bspec = pl.BlockSpec((1, 1, H), lambda t, i: (t, 0, 0))
    return pl.pallas_call(
        body,
        grid=(2, E // BE),
        in_specs=[pl.BlockSpec((1, BE, H), lambda t, i: (t, i, 0)),
                  wspec, bspec, wspec, bspec, wspec, bspec],
        out_specs=pl.BlockSpec((1, BE, H), lambda t, i: (t, i, 0)),
        out_shape=jax.ShapeDtypeStruct((2, E, H), jnp.float32),
    )(z_all, w2, b2, w3, b3, w4, b4)


def _tc_lstm(x, agg, rh, rc, wih, whh, w1s, w1d, b1, wsc):
    def body(x_ref, g_ref, h_ref, c_ref, wih_ref, whh_ref, ws_ref, wd_ref,
             b1_ref, sc_ref, h2_ref, c2_ref, a_ref, bt_ref, lg_ref):
        xb = x_ref[...]
        gates = (
            jnp.dot(xb, wih_ref[0:H], preferred_element_type=jnp.float32)
            + jnp.dot(g_ref[0], wih_ref[H:2 * H],
                      preferred_element_type=jnp.float32)
            + jnp.dot(g_ref[1], wih_ref[2 * H:3 * H],
                      preferred_element_type=jnp.float32)
            + jnp.dot(h_ref[...], whh_ref[...],
                      preferred_element_type=jnp.float32))
        i_g = gates[:, 0:H]
        f_g = gates[:, H:2 * H]
        g_g = gates[:, 2 * H:3 * H]
        o_g = gates[:, 3 * H:4 * H]
        c_new = (jax.nn.sigmoid(f_g) * c_ref[...]
                 + jax.nn.sigmoid(i_g) * jnp.tanh(g_g))
        h_new = jax.nn.sigmoid(o_g) * jnp.tanh(c_new)
        c2_ref[...] = c_new
        h2_ref[...] = h_new
        for t in range(2):
            a_ref[t] = jnp.dot(h_new, ws_ref[t],
                               preferred_element_type=jnp.float32)
            bt_ref[t] = (jnp.dot(h_new, wd_ref[t],
                                 preferred_element_type=jnp.float32)
                         + b1_ref[t])
        lg_ref[...] = jnp.sum(h_new * sc_ref[...], axis=1, keepdims=True)

    return pl.pallas_call(
        body,
        grid=(N // BN,),
        in_specs=[
            pl.BlockSpec((BN, H), lambda i: (i, 0)),
            pl.BlockSpec((2, BN, H), lambda i: (0, i, 0)),
            pl.BlockSpec((BN, H), lambda i: (i, 0)),
            pl.BlockSpec((BN, H), lambda i: (i, 0)),
            pl.BlockSpec((3 * H, 4 * H), lambda i: (0, 0)),
            pl.BlockSpec((H, 4 * H), lambda i: (0, 0)),
            pl.BlockSpec((2, H, H), lambda i: (0, 0, 0)),
            pl.BlockSpec((2, H, H), lambda i: (0, 0, 0)),
            pl.BlockSpec((2, 1, H), lambda i: (0, 0, 0)),
            pl.BlockSpec((1, H), lambda i: (0, 0)),
        ],
        out_specs=[
            pl.BlockSpec((BN, H), lambda i: (i, 0)),
            pl.BlockSpec((BN, H), lambda i: (i, 0)),
            pl.BlockSpec((2, BN, H), lambda i: (0, i, 0)),
            pl.BlockSpec((2, BN, H), lambda i: (0, i, 0)),
            pl.BlockSpec((BN, 1), lambda i: (i, 0)),
        ],
        out_shape=[
            jax.ShapeDtypeStruct((N, H), jnp.float32),
            jax.ShapeDtypeStruct((N, H), jnp.float32),
            jax.ShapeDtypeStruct((2, N, H), jnp.float32),
            jax.ShapeDtypeStruct((2, N, H), jnp.float32),
            jax.ShapeDtypeStruct((N, 1), jnp.float32),
        ],
    )(x, agg, rh, rc, wih, whh, w1s, w1d, b1, wsc)


def kernel(cell_q, edge_intra, edge_inter, params):
    p = params
    cq2 = cell_q.astype(jnp.int32).reshape(N, 1)
    ei = edge_intra.astype(jnp.int32)
    ee = edge_inter.astype(jnp.int32)
    src = jnp.concatenate([ei[0], ee[0] + N]).reshape(NW, NCH, KG)
    dst_g = jnp.concatenate([ei[1], ee[1] + N]).reshape(NW, NCH, KG)
    dst_s = jnp.concatenate([ei[1], ee[1]]).reshape(2, 16, E // 16 // KS, KS)

    w1s = jnp.stack([p['intra_Ws'][0][:H], p['inter_Ws'][0][:H]])
    w1d = jnp.stack([p['intra_Ws'][0][H:], p['inter_Ws'][0][H:]])
    b1 = jnp.stack([p['intra_bs'][0], p['inter_bs'][0]])[:, None, :]
    w2 = jnp.stack([p['intra_Ws'][1], p['inter_Ws'][1]]).astype(jnp.bfloat16)
    b2 = jnp.stack([p['intra_bs'][1], p['inter_bs'][1]])[:, None, :]
    w3 = jnp.stack([p['intra_Ws'][2], p['inter_Ws'][2]]).astype(jnp.bfloat16)
    b3 = jnp.stack([p['intra_bs'][2], p['inter_bs'][2]])[:, None, :]
    w4 = jnp.stack([p['intra_Ws'][3], p['inter_Ws'][3]]).astype(jnp.bfloat16)
    b4 = jnp.stack([p['intra_bs'][3], p['inter_bs'][3]])[:, None, :]
    zeros_nh = jnp.zeros((N, H), jnp.float32)

    x, A, B = _tc_init(cq2, p['digit_embed'], w1s, w1d, b1)
    rh = zeros_nh
    rc = zeros_nh
    lg = None
    for _ in range(STEPS):
        z = _sc_gather_add(A.reshape(2 * N, H), B.reshape(2 * N, H),
                           src, dst_g).reshape(2, E, H)
        m = _tc_mlp(z, w2, b2, w3, b3, w4, b4)
        agg = _sc_scatter_add(m, dst_s, zeros_nh)
        rh, rc, A, B, lg = _tc_lstm(x, agg, rh, rc, p['W_ih'], p['W_hh'],
                                    w1s, w1d, b1, p['w_score'][None, :])
    return lg[:, 0]


# speedup vs baseline: 1.5886x; 1.5886x over previous
import functools

import jax
import jax.numpy as jnp
from jax import lax
from jax.experimental import pallas as pl
from jax.experimental.pallas import tpu as pltpu
from jax.experimental.pallas import tpu_sc as plsc

N = 10000
H = 128
E = 160000
STEPS = 4

NW = 32
EW = 2 * E // NW
KG = 80
ES = E // 16
KS = 80
NSR = 624
NTAIL = N - 16 * NSR
BN = 1000
BE = 2000

_SC_MESH = dict(core_axis_name="c", subcore_axis_name="s")


NCH = EW // KG
NPAIR = (NCH - 1) // 2


def _sc_gather_add(tab_a, tab_b, src, dst):
    @functools.partial(
        pl.kernel,
        mesh=plsc.VectorSubcoreMesh(**_SC_MESH),
        out_type=jax.ShapeDtypeStruct((2 * E, H), jnp.float32),
        scratch_types=[
            pltpu.VMEM((NCH, KG), jnp.int32),
            pltpu.VMEM((NCH, KG), jnp.int32),
            pltpu.VMEM((KG, H), jnp.float32),
            pltpu.VMEM((KG, H), jnp.float32),
            pltpu.VMEM((KG, H), jnp.float32),
            pltpu.VMEM((KG, H), jnp.float32),
            pltpu.SemaphoreType.DMA,
            pltpu.SemaphoreType.DMA,
            pltpu.SemaphoreType.DMA,
            pltpu.SemaphoreType.DMA,
        ],
    )
    def k(ta, tb, s_idx, d_idx, z_out, si, di,
          ba0, bb0, ba1, bb1, g0, g1, w0, w1):
        wid = lax.axis_index("s") * 2 + lax.axis_index("c")
        base = pl.multiple_of(wid * EW, 8)
        pltpu.sync_copy(s_idx.at[wid], si)
        pltpu.sync_copy(d_idx.at[wid], di)

        def gdesc(i, ba, bb, sem):
            return (pltpu.make_async_copy(ta.at[si.at[i]], ba, sem),
                    pltpu.make_async_copy(tb.at[di.at[i]], bb, sem))

        def wdesc(i, ba, sem):
            off = pl.multiple_of(base + i * KG, 8)
            return pltpu.make_async_copy(ba, z_out.at[pl.ds(off, KG)], sem)

        def fire(i, ba, bb, sem):
            for d in gdesc(i, ba, bb, sem):
                d.start()

        def wait_gather(i, ba, bb, sem):
            for d in gdesc(i, ba, bb, sem):
                d.wait()

        def add_pair(ba, bb):
            def addrow(r, carry):
                for c in range(H // 16):
                    sl = pl.ds(c * 16, 16)
                    plsc.addupdate(ba.at[r, sl], bb[r, sl])
                return carry

            lax.fori_loop(0, KG, addrow, 0)

        fire(0, ba0, bb0, g0)
        fire(1, ba1, bb1, g1)

        def body(g, carry):
            i0 = 2 * g
            wait_gather(i0, ba0, bb0, g0)
            add_pair(ba0, bb0)
            wdesc(i0, ba0, w0).start()
            wait_gather(i0 + 1, ba1, bb1, g1)
            add_pair(ba1, bb1)
            wdesc(i0 + 1, ba1, w1).start()
            wdesc(i0, ba0, w0).wait()
            fire(i0 + 2, ba0, bb0, g0)

            @pl.when(g < NPAIR - 1)
            def _refill():
                wdesc(i0 + 1, ba1, w1).wait()
                fire(i0 + 3, ba1, bb1, g1)

            return carry

        lax.fori_loop(0, NPAIR, body, 0)
        wait_gather(NCH - 1, ba0, bb0, g0)
        add_pair(ba0, bb0)
        wdesc(NCH - 1, ba0, w0).start()
        wdesc(NCH - 1, ba0, w0).wait()
        wdesc(NCH - 2, ba1, w1).wait()

    return k(tab_a, tab_b, src, dst)


def _sc_scatter_add(m_all, dst2, zeros_nh):
    NCS = ES // KS

    @functools.partial(
        pl.kernel,
        mesh=plsc.VectorSubcoreMesh(**_SC_MESH),
        out_type=jax.ShapeDtypeStruct((2, N, H), jnp.float32),
        scratch_types=[
            pltpu.VMEM((NCS, KS), jnp.int32),
            pltpu.VMEM((KS, H), jnp.float32),
            pltpu.VMEM((KS, H), jnp.float32),
            pltpu.VMEM_SHARED((N, H), jnp.float32),
            pltpu.SemaphoreType.DMA,
            pltpu.SemaphoreType.DMA,
        ],
    )
    def k(m_hbm, d_idx, z_hbm, agg_out, idx2, m0, m1, agg_sh, r0s, r1s):
        c = lax.axis_index("c")
        s = lax.axis_index("s")
        row0 = pl.multiple_of(s * NSR, 8)

        def rdesc(i, buf, sem):
            off = pl.multiple_of(s * ES + i * KS, 8)
            return pltpu.make_async_copy(m_hbm.at[c, pl.ds(off, KS)],
                                         buf, sem)

        pltpu.sync_copy(d_idx.at[c, s], idx2)
        rdesc(0, m0, r0s).start()
        rdesc(1, m1, r1s).start()
        pltpu.sync_copy(z_hbm.at[pl.ds(row0, NSR)],
                        agg_sh.at[pl.ds(row0, NSR)])

        @pl.when(s == 15)
        def _zero_tail():
            pltpu.sync_copy(z_hbm.at[pl.ds(16 * NSR, NTAIL)],
                            agg_sh.at[pl.ds(16 * NSR, NTAIL)])

        plsc.subcore_barrier()

        def body(g, carry):
            i0 = 2 * g
            rdesc(i0, m0, r0s).wait()
            pltpu.sync_copy(m0, agg_sh.at[idx2.at[i0]], add=True)
            rdesc(i0 + 2, m0, r0s).start()
            rdesc(i0 + 1, m1, r1s).wait()
            pltpu.sync_copy(m1, agg_sh.at[idx2.at[i0 + 1]], add=True)

            @pl.when(g < NPAIR - 1)
            def _refill():
                rdesc(i0 + 3, m1, r1s).start()

            return carry

        lax.fori_loop(0, (NCS - 1) // 2, body, 0)
        rdesc(NCS - 1, m0, r0s).wait()
        pltpu.sync_copy(m0, agg_sh.at[idx2.at[NCS - 1]], add=True)
        plsc.subcore_barrier()
        pltpu.sync_copy(agg_sh.at[pl.ds(row0, NSR)],
                        agg_out.at[c, pl.ds(row0, NSR)])

        @pl.when(s == 15)
        def _write_tail():
            pltpu.sync_copy(agg_sh.at[pl.ds(16 * NSR, NTAIL)],
                            agg_out.at[c, pl.ds(16 * NSR, NTAIL)])

    return k(m_all, dst2, zeros_nh)


def _tc_init(cq2, emb, w1s, w1d, b1):
    def body(q_ref, e_ref, ws_ref, wd_ref, b1_ref, x_ref, a_ref, bt_ref):
        q = q_ref[...]
        e = e_ref[...]
        x = jnp.where(q == 0, e[0:1, :], jnp.where(q == 1, e[1:2, :], e[2:3, :]))
        x_ref[...] = x
        for t in range(2):
            a_ref[t] = jnp.dot(x, ws_ref[t], preferred_element_type=jnp.float32)
            bt_ref[t] = (jnp.dot(x, wd_ref[t],
                                 preferred_element_type=jnp.float32)
                         + b1_ref[t])

    return pl.pallas_call(
        body,
        grid=(N // BN,),
        in_specs=[
            pl.BlockSpec((BN, 1), lambda i: (i, 0)),
            pl.BlockSpec((3, H), lambda i: (0, 0)),
            pl.BlockSpec((2, H, H), lambda i: (0, 0, 0)),
            pl.BlockSpec((2, H, H), lambda i: (0, 0, 0)),
            pl.BlockSpec((2, 1, H), lambda i: (0, 0, 0)),
        ],
        out_specs=[
            pl.BlockSpec((BN, H), lambda i: (i, 0)),
            pl.BlockSpec((2, BN, H), lambda i: (0, i, 0)),
            pl.BlockSpec((2, BN, H), lambda i: (0, i, 0)),
        ],
        out_shape=[
            jax.ShapeDtypeStruct((N, H), jnp.float32),
            jax.ShapeDtypeStruct((2, N, H), jnp.float32),
            jax.ShapeDtypeStruct((2, N, H), jnp.float32),
        ],
    )(cq2, emb, w1s, w1d, b1)


def _tc_mlp(z_all, w2, b2, w3, b3, w4, b4):
    def body(z_ref, w2r, b2r, w3r, b3r, w4r, b4r, m_ref):
        t = jnp.maximum(z_ref[0], 0.0).astype(jnp.bfloat16)
        t = jnp.maximum(
            jnp.dot(t, w2r[0], preferred_element_type=jnp.float32) + b2r[0],
            0.0).astype(jnp.bfloat16)
        t = jnp.maximum(
            jnp.dot(t, w3r[0], preferred_element_type=jnp.float32) + b3r[0],
            0.0).astype(jnp.bfloat16)
        m_ref[0] = jnp.dot(t, w4r[0], preferred_element_type=jnp.float32) + b4r[0]

    wspec = pl.BlockSpec((1, H, H), lambda t, i: (t, 0, 0))
    bspec = pl.BlockSpec((1, 1, H), lambda t, i: (t, 0, 0))
    return pl.pallas_call(
        body,
        grid=(2, E // BE),
        in_specs=[pl.BlockSpec((1, BE, H), lambda t, i: (t, i, 0)),
                  wspec, bspec, wspec, bspec, wspec, bspec],
        out_specs=pl.BlockSpec((1, BE, H), lambda t, i: (t, i, 0)),
        out_shape=jax.ShapeDtypeStruct((2, E, H), jnp.float32),
    )(z_all, w2, b2, w3, b3, w4, b4)


def _tc_lstm(x, agg, rh, rc, wih, whh, w1s, w1d, b1, wsc):
    def body(x_ref, g_ref, h_ref, c_ref, wih_ref, whh_ref, ws_ref, wd_ref,
             b1_ref, sc_ref, h2_ref, c2_ref, a_ref, bt_ref, lg_ref):
        xb = x_ref[...]
        gates = (
            jnp.dot(xb, wih_ref[0:H], preferred_element_type=jnp.float32)
            + jnp.dot(g_ref[0], wih_ref[H:2 * H],
                      preferred_element_type=jnp.float32)
            + jnp.dot(g_ref[1], wih_ref[2 * H:3 * H],
                      preferred_element_type=jnp.float32)
            + jnp.dot(h_ref[...], whh_ref[...],
                      preferred_element_type=jnp.float32))
        i_g = gates[:, 0:H]
        f_g = gates[:, H:2 * H]
        g_g = gates[:, 2 * H:3 * H]
        o_g = gates[:, 3 * H:4 * H]
        c_new = (jax.nn.sigmoid(f_g) * c_ref[...]
                 + jax.nn.sigmoid(i_g) * jnp.tanh(g_g))
        h_new = jax.nn.sigmoid(o_g) * jnp.tanh(c_new)
        c2_ref[...] = c_new
        h2_ref[...] = h_new
        for t in range(2):
            a_ref[t] = jnp.dot(h_new, ws_ref[t],
                               preferred_element_type=jnp.float32)
            bt_ref[t] = (jnp.dot(h_new, wd_ref[t],
                                 preferred_element_type=jnp.float32)
                         + b1_ref[t])
        lg_ref[...] = jnp.sum(h_new * sc_ref[...], axis=1, keepdims=True)

    return pl.pallas_call(
        body,
        grid=(N // BN,),
        in_specs=[
            pl.BlockSpec((BN, H), lambda i: (i, 0)),
            pl.BlockSpec((2, BN, H), lambda i: (0, i, 0)),
            pl.BlockSpec((BN, H), lambda i: (i, 0)),
            pl.BlockSpec((BN, H), lambda i: (i, 0)),
            pl.BlockSpec((3 * H, 4 * H), lambda i: (0, 0)),
            pl.BlockSpec((H, 4 * H), lambda i: (0, 0)),
            pl.BlockSpec((2, H, H), lambda i: (0, 0, 0)),
            pl.BlockSpec((2, H, H), lambda i: (0, 0, 0)),
            pl.BlockSpec((2, 1, H), lambda i: (0, 0, 0)),
            pl.BlockSpec((1, H), lambda i: (0, 0)),
        ],
        out_specs=[
            pl.BlockSpec((BN, H), lambda i: (i, 0)),
            pl.BlockSpec((BN, H), lambda i: (i, 0)),
            pl.BlockSpec((2, BN, H), lambda i: (0, i, 0)),
            pl.BlockSpec((2, BN, H), lambda i: (0, i, 0)),
            pl.BlockSpec((BN, 1), lambda i: (i, 0)),
        ],
        out_shape=[
            jax.ShapeDtypeStruct((N, H), jnp.float32),
            jax.ShapeDtypeStruct((N, H), jnp.float32),
            jax.ShapeDtypeStruct((2, N, H), jnp.float32),
            jax.ShapeDtypeStruct((2, N, H), jnp.float32),
            jax.ShapeDtypeStruct((N, 1), jnp.float32),
        ],
    )(x, agg, rh, rc, wih, whh, w1s, w1d, b1, wsc)


def kernel(cell_q, edge_intra, edge_inter, params):
    p = params
    cq2 = cell_q.astype(jnp.int32).reshape(N, 1)
    ei = edge_intra.astype(jnp.int32)
    ee = edge_inter.astype(jnp.int32)
    src = jnp.concatenate([ei[0], ee[0] + N]).reshape(NW, NCH, KG)
    dst_g = jnp.concatenate([ei[1], ee[1] + N]).reshape(NW, NCH, KG)
    dst_s = jnp.concatenate([ei[1], ee[1]]).reshape(2, 16, E // 16 // KS, KS)

    w1s = jnp.stack([p['intra_Ws'][0][:H], p['inter_Ws'][0][:H]])
    w1d = jnp.stack([p['intra_Ws'][0][H:], p['inter_Ws'][0][H:]])
    b1 = jnp.stack([p['intra_bs'][0], p['inter_bs'][0]])[:, None, :]
    w2 = jnp.stack([p['intra_Ws'][1], p['inter_Ws'][1]]).astype(jnp.bfloat16)
    b2 = jnp.stack([p['intra_bs'][1], p['inter_bs'][1]])[:, None, :]
    w3 = jnp.stack([p['intra_Ws'][2], p['inter_Ws'][2]]).astype(jnp.bfloat16)
    b3 = jnp.stack([p['intra_bs'][2], p['inter_bs'][2]])[:, None, :]
    w4 = jnp.stack([p['intra_Ws'][3], p['inter_Ws'][3]]).astype(jnp.bfloat16)
    b4 = jnp.stack([p['intra_bs'][3], p['inter_bs'][3]])[:, None, :]
    zeros_nh = jnp.zeros((N, H), jnp.float32)

    x, A, B = _tc_init(cq2, p['digit_embed'], w1s, w1d, b1)
    rh = zeros_nh
    rc = zeros_nh
    lg = None
    for _ in range(STEPS):
        z = _sc_gather_add(A.reshape(2 * N, H), B.reshape(2 * N, H),
                           src, dst_g).reshape(2, E, H)
        m = _tc_mlp(z, w2, b2, w3, b3, w4, b4)
        agg = _sc_scatter_add(m, dst_s, zeros_nh)
        rh, rc, A, B, lg = _tc_lstm(x, agg, rh, rc, p['W_ih'], p['W_hh'],
                                    w1s, w1d, b1, p['w_score'][None, :])
    return lg[:, 0]


# speedup vs baseline: 1.7793x; 1.1201x over previous
import functools

import jax
import jax.numpy as jnp
from jax import lax
from jax.experimental import pallas as pl
from jax.experimental.pallas import tpu as pltpu
from jax.experimental.pallas import tpu_sc as plsc

N = 10000
H = 128
E = 160000
STEPS = 4

NW = 32
EW = 2 * E // NW
KG = 80
ES = E // 16
KS = 80
NSR = 624
NTAIL = N - 16 * NSR
BN = 1000
BE = 2000

_SC_MESH = dict(core_axis_name="c", subcore_axis_name="s")


NCH = EW // KG
NPAIR = (NCH - 1) // 2


def _sc_gather_add(tab_a, tab_b, src, dst):
    _, NCH, KG = src.shape
    EW = NCH * KG
    NPAIR = (NCH - 1) // 2

    @functools.partial(
        pl.kernel,
        mesh=plsc.VectorSubcoreMesh(**_SC_MESH),
        out_type=jax.ShapeDtypeStruct((NW * EW, H), jnp.float32),
        scratch_types=[
            pltpu.VMEM((NCH, KG), jnp.int32),
            pltpu.VMEM((NCH, KG), jnp.int32),
            pltpu.VMEM((KG, H), jnp.float32),
            pltpu.VMEM((KG, H), jnp.float32),
            pltpu.VMEM((KG, H), jnp.float32),
            pltpu.VMEM((KG, H), jnp.float32),
            pltpu.SemaphoreType.DMA,
            pltpu.SemaphoreType.DMA,
            pltpu.SemaphoreType.DMA,
            pltpu.SemaphoreType.DMA,
        ],
    )
    def k(ta, tb, s_idx, d_idx, z_out, si, di,
          ba0, bb0, ba1, bb1, g0, g1, w0, w1):
        wid = lax.axis_index("s") * 2 + lax.axis_index("c")
        base = pl.multiple_of(wid * EW, 8)
        pltpu.sync_copy(s_idx.at[wid], si)
        pltpu.sync_copy(d_idx.at[wid], di)

        def gdesc(i, ba, bb, sem):
            return (pltpu.make_async_copy(ta.at[si.at[i]], ba, sem),
                    pltpu.make_async_copy(tb.at[di.at[i]], bb, sem))

        def wdesc(i, ba, sem):
            off = pl.multiple_of(base + i * KG, 8)
            return pltpu.make_async_copy(ba, z_out.at[pl.ds(off, KG)], sem)

        def fire(i, ba, bb, sem):
            for d in gdesc(i, ba, bb, sem):
                d.start()

        def wait_gather(i, ba, bb, sem):
            for d in gdesc(i, ba, bb, sem):
                d.wait()

        def add_pair(ba, bb):
            def addrow(r, carry):
                for c in range(H // 16):
                    sl = pl.ds(c * 16, 16)
                    plsc.addupdate(ba.at[r, sl], bb[r, sl])
                return carry

            lax.fori_loop(0, KG, addrow, 0)

        fire(0, ba0, bb0, g0)
        fire(1, ba1, bb1, g1)

        def body(g, carry):
            i0 = 2 * g
            wait_gather(i0, ba0, bb0, g0)
            add_pair(ba0, bb0)
            wdesc(i0, ba0, w0).start()
            wait_gather(i0 + 1, ba1, bb1, g1)
            add_pair(ba1, bb1)
            wdesc(i0 + 1, ba1, w1).start()
            wdesc(i0, ba0, w0).wait()
            fire(i0 + 2, ba0, bb0, g0)

            @pl.when(g < NPAIR - 1)
            def _refill():
                wdesc(i0 + 1, ba1, w1).wait()
                fire(i0 + 3, ba1, bb1, g1)

            return carry

        lax.fori_loop(0, NPAIR, body, 0)
        wait_gather(NCH - 1, ba0, bb0, g0)
        add_pair(ba0, bb0)
        wdesc(NCH - 1, ba0, w0).start()
        wdesc(NCH - 1, ba0, w0).wait()
        wdesc(NCH - 2, ba1, w1).wait()

    return k(tab_a, tab_b, src, dst)


def _sc_scatter_add(m_all, dst2, zeros_nh):
    _, _, NCS, KS2 = dst2.shape
    ES2 = NCS * KS2
    NPAIR = (NCS - 1) // 2

    @functools.partial(
        pl.kernel,
        mesh=plsc.VectorSubcoreMesh(**_SC_MESH),
        out_type=jax.ShapeDtypeStruct((2, N, H), jnp.float32),
        scratch_types=[
            pltpu.VMEM((NCS, KS2), jnp.int32),
            pltpu.VMEM((KS2, H), jnp.float32),
            pltpu.VMEM((KS2, H), jnp.float32),
            pltpu.VMEM_SHARED((N, H), jnp.float32),
            pltpu.SemaphoreType.DMA,
            pltpu.SemaphoreType.DMA,
        ],
    )
    def k(m_hbm, d_idx, z_hbm, agg_out, idx2, m0, m1, agg_sh, r0s, r1s):
        c = lax.axis_index("c")
        s = lax.axis_index("s")
        row0 = pl.multiple_of(s * NSR, 8)

        def rdesc(i, buf, sem):
            off = pl.multiple_of(s * ES2 + i * KS2, 8)
            return pltpu.make_async_copy(m_hbm.at[c, pl.ds(off, KS2)],
                                         buf, sem)

        pltpu.sync_copy(d_idx.at[c, s], idx2)
        rdesc(0, m0, r0s).start()
        rdesc(1, m1, r1s).start()
        pltpu.sync_copy(z_hbm.at[pl.ds(row0, NSR)],
                        agg_sh.at[pl.ds(row0, NSR)])

        @pl.when(s == 15)
        def _zero_tail():
            pltpu.sync_copy(z_hbm.at[pl.ds(16 * NSR, NTAIL)],
                            agg_sh.at[pl.ds(16 * NSR, NTAIL)])

        plsc.subcore_barrier()

        def body(g, carry):
            i0 = 2 * g
            rdesc(i0, m0, r0s).wait()
            pltpu.sync_copy(m0, agg_sh.at[idx2.at[i0]], add=True)
            rdesc(i0 + 2, m0, r0s).start()
            rdesc(i0 + 1, m1, r1s).wait()
            pltpu.sync_copy(m1, agg_sh.at[idx2.at[i0 + 1]], add=True)

            @pl.when(g < NPAIR - 1)
            def _refill():
                rdesc(i0 + 3, m1, r1s).start()

            return carry

        lax.fori_loop(0, (NCS - 1) // 2, body, 0)
        rdesc(NCS - 1, m0, r0s).wait()
        pltpu.sync_copy(m0, agg_sh.at[idx2.at[NCS - 1]], add=True)
        plsc.subcore_barrier()
        pltpu.sync_copy(agg_sh.at[pl.ds(row0, NSR)],
                        agg_out.at[c, pl.ds(row0, NSR)])

        @pl.when(s == 15)
        def _write_tail():
            pltpu.sync_copy(agg_sh.at[pl.ds(16 * NSR, NTAIL)],
                            agg_out.at[c, pl.ds(16 * NSR, NTAIL)])

    return k(m_all, dst2, zeros_nh)


def _tc_init(cq2, emb, w1s, w1d, b1):
    def body(q_ref, e_ref, ws_ref, wd_ref, b1_ref, x_ref, a_ref, bt_ref):
        q = q_ref[...]
        e = e_ref[...]
        x = jnp.where(q == 0, e[0:1, :], jnp.where(q == 1, e[1:2, :], e[2:3, :]))
        x_ref[...] = x
        for t in range(2):
            a_ref[t] = jnp.dot(x, ws_ref[t], preferred_element_type=jnp.float32)
            bt_ref[t] = (jnp.dot(x, wd_ref[t],
                                 preferred_element_type=jnp.float32)
                         + b1_ref[t])

    return pl.pallas_call(
        body,
        grid=(N // BN,),
        in_specs=[
            pl.BlockSpec((BN, 1), lambda i: (i, 0)),
            pl.BlockSpec((3, H), lambda i: (0, 0)),
            pl.BlockSpec((2, H, H), lambda i: (0, 0, 0)),
            pl.BlockSpec((2, H, H), lambda i: (0, 0, 0)),
            pl.BlockSpec((2, 1, H), lambda i: (0, 0, 0)),
        ],
        out_specs=[
            pl.BlockSpec((BN, H), lambda i: (i, 0)),
            pl.BlockSpec((2, BN, H), lambda i: (0, i, 0)),
            pl.BlockSpec((2, BN, H), lambda i: (0, i, 0)),
        ],
        out_shape=[
            jax.ShapeDtypeStruct((N, H), jnp.float32),
            jax.ShapeDtypeStruct((2, N, H), jnp.float32),
            jax.ShapeDtypeStruct((2, N, H), jnp.float32),
        ],
    )(cq2, emb, w1s, w1d, b1)


def _tc_mlp(z_all, w2, b2, w3, b3, w4, b4):
    def body(z_ref, w2r, b2r, w3r, b3r, w4r, b4r, m_ref):
        t = jnp.maximum(z_ref[0], 0.0).astype(jnp.bfloat16)
        t = jnp.maximum(
            jnp.dot(t, w2r[0], preferred_element_type=jnp.float32) + b2r[0],
            0.0).astype(jnp.bfloat16)
        t = jnp.maximum(
            jnp.dot(t, w3r[0], preferred_element_type=jnp.float32) + b3r[0],
            0.0).astype(jnp.bfloat16)
        m_ref[0] = jnp.dot(t, w4r[0], preferred_element_type=jnp.float32) + b4r[0]

    ne = z_all.shape[1]
    wspec = pl.BlockSpec((1, H, H), lambda t, i: (t, 0, 0))
    bspec = pl.BlockSpec((1, 1, H), lambda t, i: (t, 0, 0))
    return pl.pallas_call(
        body,
        grid=(2, ne // BE),
        in_specs=[pl.BlockSpec((1, BE, H), lambda t, i: (t, i, 0)),
                  wspec, bspec, wspec, bspec, wspec, bspec],
        out_specs=pl.BlockSpec((1, BE, H), lambda t, i: (t, i, 0)),
        out_shape=jax.ShapeDtypeStruct((2, ne, H), jnp.float32),
    )(z_all, w2, b2, w3, b3, w4, b4)


def _tc_lstm(x, agg0, agg1, rh, rc, wih, whh, w1s, w1d, b1, wsc):
    def body(x_ref, g0_ref, g1_ref, h_ref, c_ref, wih_ref, whh_ref, ws_ref,
             wd_ref, b1_ref, sc_ref, h2_ref, c2_ref, a_ref, bt_ref, lg_ref):
        xb = x_ref[...]
        gates = (
            jnp.dot(xb, wih_ref[0:H], preferred_element_type=jnp.float32)
            + jnp.dot(g0_ref[0] + g1_ref[0], wih_ref[H:2 * H],
                      preferred_element_type=jnp.float32)
            + jnp.dot(g0_ref[1] + g1_ref[1], wih_ref[2 * H:3 * H],
                      preferred_element_type=jnp.float32)
            + jnp.dot(h_ref[...], whh_ref[...],
                      preferred_element_type=jnp.float32))
        i_g = gates[:, 0:H]
        f_g = gates[:, H:2 * H]
        g_g = gates[:, 2 * H:3 * H]
        o_g = gates[:, 3 * H:4 * H]
        c_new = (jax.nn.sigmoid(f_g) * c_ref[...]
                 + jax.nn.sigmoid(i_g) * jnp.tanh(g_g))
        h_new = jax.nn.sigmoid(o_g) * jnp.tanh(c_new)
        c2_ref[...] = c_new
        h2_ref[...] = h_new
        for t in range(2):
            a_ref[t] = jnp.dot(h_new, ws_ref[t],
                               preferred_element_type=jnp.float32)
            bt_ref[t] = (jnp.dot(h_new, wd_ref[t],
                                 preferred_element_type=jnp.float32)
                         + b1_ref[t])
        lg_ref[...] = jnp.sum(h_new * sc_ref[...], axis=1, keepdims=True)

    return pl.pallas_call(
        body,
        grid=(N // BN,),
        in_specs=[
            pl.BlockSpec((BN, H), lambda i: (i, 0)),
            pl.BlockSpec((2, BN, H), lambda i: (0, i, 0)),
            pl.BlockSpec((2, BN, H), lambda i: (0, i, 0)),
            pl.BlockSpec((BN, H), lambda i: (i, 0)),
            pl.BlockSpec((BN, H), lambda i: (i, 0)),
            pl.BlockSpec((3 * H, 4 * H), lambda i: (0, 0)),
            pl.BlockSpec((H, 4 * H), lambda i: (0, 0)),
            pl.BlockSpec((2, H, H), lambda i: (0, 0, 0)),
            pl.BlockSpec((2, H, H), lambda i: (0, 0, 0)),
            pl.BlockSpec((2, 1, H), lambda i: (0, 0, 0)),
            pl.BlockSpec((1, H), lambda i: (0, 0)),
        ],
        out_specs=[
            pl.BlockSpec((BN, H), lambda i: (i, 0)),
            pl.BlockSpec((BN, H), lambda i: (i, 0)),
            pl.BlockSpec((2, BN, H), lambda i: (0, i, 0)),
            pl.BlockSpec((2, BN, H), lambda i: (0, i, 0)),
            pl.BlockSpec((BN, 1), lambda i: (i, 0)),
        ],
        out_shape=[
            jax.ShapeDtypeStruct((N, H), jnp.float32),
            jax.ShapeDtypeStruct((N, H), jnp.float32),
            jax.ShapeDtypeStruct((2, N, H), jnp.float32),
            jax.ShapeDtypeStruct((2, N, H), jnp.float32),
            jax.ShapeDtypeStruct((N, 1), jnp.float32),
        ],
    )(x, agg0, agg1, rh, rc, wih, whh, w1s, w1d, b1, wsc)


def kernel(cell_q, edge_intra, edge_inter, params):
    p = params
    cq2 = cell_q.astype(jnp.int32).reshape(N, 1)
    ei = edge_intra.astype(jnp.int32)
    ee = edge_inter.astype(jnp.int32)
    Eh = E // 2
    KGH, NCHH = 40, 125

    def _half(a_i, a_e, h, off):
        sl = slice(h * Eh, (h + 1) * Eh)
        return jnp.concatenate([a_i[sl], a_e[sl] + off])

    src_h = [_half(ei[0], ee[0], h, N).reshape(NW, NCHH, KGH) for h in (0, 1)]
    dst_gh = [_half(ei[1], ee[1], h, N).reshape(NW, NCHH, KGH) for h in (0, 1)]
    dst_sh = [_half(ei[1], ee[1], h, 0).reshape(2, 16, NCHH, KGH)
              for h in (0, 1)]

    w1s = jnp.stack([p['intra_Ws'][0][:H], p['inter_Ws'][0][:H]])
    w1d = jnp.stack([p['intra_Ws'][0][H:], p['inter_Ws'][0][H:]])
    b1 = jnp.stack([p['intra_bs'][0], p['inter_bs'][0]])[:, None, :]
    w2 = jnp.stack([p['intra_Ws'][1], p['inter_Ws'][1]]).astype(jnp.bfloat16)
    b2 = jnp.stack([p['intra_bs'][1], p['inter_bs'][1]])[:, None, :]
    w3 = jnp.stack([p['intra_Ws'][2], p['inter_Ws'][2]]).astype(jnp.bfloat16)
    b3 = jnp.stack([p['intra_bs'][2], p['inter_bs'][2]])[:, None, :]
    w4 = jnp.stack([p['intra_Ws'][3], p['inter_Ws'][3]]).astype(jnp.bfloat16)
    b4 = jnp.stack([p['intra_bs'][3], p['inter_bs'][3]])[:, None, :]
    zeros_nh = jnp.zeros((N, H), jnp.float32)

    x, A, B = _tc_init(cq2, p['digit_embed'], w1s, w1d, b1)
    rh = zeros_nh
    rc = zeros_nh
    lg = None
    for _ in range(STEPS):
        A2 = A.reshape(2 * N, H)
        B2 = B.reshape(2 * N, H)
        z0 = _sc_gather_add(A2, B2, src_h[0], dst_gh[0]).reshape(2, Eh, H)
        z1 = _sc_gather_add(A2, B2, src_h[1], dst_gh[1]).reshape(2, Eh, H)
        m0 = _tc_mlp(z0, w2, b2, w3, b3, w4, b4)
        m1 = _tc_mlp(z1, w2, b2, w3, b3, w4, b4)
        ag0 = _sc_scatter_add(m0, dst_sh[0], zeros_nh)
        ag1 = _sc_scatter_add(m1, dst_sh[1], zeros_nh)
        rh, rc, A, B, lg = _tc_lstm(x, ag0, ag1, rh, rc, p['W_ih'],
                                    p['W_hh'], w1s, w1d, b1,
                                    p['w_score'][None, :])
    return lg[:, 0]


# speedup vs baseline: 1.8852x; 1.0595x over previous
import functools

import jax
import jax.numpy as jnp
from jax import lax
from jax.experimental import pallas as pl
from jax.experimental.pallas import tpu as pltpu
from jax.experimental.pallas import tpu_sc as plsc

N = 10000
H = 128
E = 160000
STEPS = 4

NW = 32
EW = 2 * E // NW
KG = 80
ES = E // 16
KS = 80
NSR = 624
NTAIL = N - 16 * NSR
BN = 1000
BE = 2000

_SC_MESH = dict(core_axis_name="c", subcore_axis_name="s")


NCH = EW // KG
NPAIR = (NCH - 1) // 2


def _sc_gather_add(tab_a, tab_b, src, dst):
    _, NCH, KG = src.shape
    EW = NCH * KG
    NPAIR = (NCH - 1) // 2

    @functools.partial(
        pl.kernel,
        mesh=plsc.VectorSubcoreMesh(**_SC_MESH),
        out_type=jax.ShapeDtypeStruct((NW * EW, H), jnp.float32),
        scratch_types=[
            pltpu.VMEM((NCH, KG), jnp.int32),
            pltpu.VMEM((NCH, KG), jnp.int32),
            pltpu.VMEM((KG, H), jnp.float32),
            pltpu.VMEM((KG, H), jnp.float32),
            pltpu.VMEM((KG, H), jnp.float32),
            pltpu.VMEM((KG, H), jnp.float32),
            pltpu.SemaphoreType.DMA,
            pltpu.SemaphoreType.DMA,
            pltpu.SemaphoreType.DMA,
            pltpu.SemaphoreType.DMA,
        ],
    )
    def k(ta, tb, s_idx, d_idx, z_out, si, di,
          ba0, bb0, ba1, bb1, g0, g1, w0, w1):
        wid = lax.axis_index("s") * 2 + lax.axis_index("c")
        base = pl.multiple_of(wid * EW, 8)
        pltpu.sync_copy(s_idx.at[wid], si)
        pltpu.sync_copy(d_idx.at[wid], di)

        def gdesc(i, ba, bb, sem):
            return (pltpu.make_async_copy(ta.at[si.at[i]], ba, sem),
                    pltpu.make_async_copy(tb.at[di.at[i]], bb, sem))

        def wdesc(i, ba, sem):
            off = pl.multiple_of(base + i * KG, 8)
            return pltpu.make_async_copy(ba, z_out.at[pl.ds(off, KG)], sem)

        def fire(i, ba, bb, sem):
            for d in gdesc(i, ba, bb, sem):
                d.start()

        def wait_gather(i, ba, bb, sem):
            for d in gdesc(i, ba, bb, sem):
                d.wait()

        def add_pair(ba, bb):
            def addrow(r, carry):
                for c in range(H // 16):
                    sl = pl.ds(c * 16, 16)
                    plsc.addupdate(ba.at[r, sl], bb[r, sl])
                return carry

            lax.fori_loop(0, KG, addrow, 0)

        fire(0, ba0, bb0, g0)
        fire(1, ba1, bb1, g1)

        def body(g, carry):
            i0 = 2 * g
            wait_gather(i0, ba0, bb0, g0)
            add_pair(ba0, bb0)
            wdesc(i0, ba0, w0).start()
            wait_gather(i0 + 1, ba1, bb1, g1)
            add_pair(ba1, bb1)
            wdesc(i0 + 1, ba1, w1).start()
            wdesc(i0, ba0, w0).wait()
            fire(i0 + 2, ba0, bb0, g0)

            @pl.when(g < NPAIR - 1)
            def _refill():
                wdesc(i0 + 1, ba1, w1).wait()
                fire(i0 + 3, ba1, bb1, g1)

            return carry

        lax.fori_loop(0, NPAIR, body, 0)
        if NCH % 2 == 1:
            wait_gather(NCH - 1, ba0, bb0, g0)
            add_pair(ba0, bb0)
            wdesc(NCH - 1, ba0, w0).start()
            wdesc(NCH - 1, ba0, w0).wait()
            wdesc(NCH - 2, ba1, w1).wait()
        else:
            wait_gather(NCH - 2, ba0, bb0, g0)
            add_pair(ba0, bb0)
            wdesc(NCH - 2, ba0, w0).start()
            wdesc(NCH - 3, ba1, w1).wait()
            fire(NCH - 1, ba1, bb1, g1)
            wait_gather(NCH - 1, ba1, bb1, g1)
            add_pair(ba1, bb1)
            wdesc(NCH - 1, ba1, w1).start()
            wdesc(NCH - 2, ba0, w0).wait()
            wdesc(NCH - 1, ba1, w1).wait()

    return k(tab_a, tab_b, src, dst)


def _sc_scatter_add(m_all, dst2, zeros_nh):
    _, _, NCS, KS2 = dst2.shape
    ES2 = NCS * KS2
    NPAIR = (NCS - 1) // 2

    @functools.partial(
        pl.kernel,
        mesh=plsc.VectorSubcoreMesh(**_SC_MESH),
        out_type=jax.ShapeDtypeStruct((2, N, H), jnp.float32),
        scratch_types=[
            pltpu.VMEM((NCS, KS2), jnp.int32),
            pltpu.VMEM((KS2, H), jnp.float32),
            pltpu.VMEM((KS2, H), jnp.float32),
            pltpu.VMEM_SHARED((N, H), jnp.float32),
            pltpu.SemaphoreType.DMA,
            pltpu.SemaphoreType.DMA,
        ],
    )
    def k(m_hbm, d_idx, z_hbm, agg_out, idx2, m0, m1, agg_sh, r0s, r1s):
        c = lax.axis_index("c")
        s = lax.axis_index("s")
        row0 = pl.multiple_of(s * NSR, 8)

        def rdesc(i, buf, sem):
            off = pl.multiple_of(s * ES2 + i * KS2, 8)
            return pltpu.make_async_copy(m_hbm.at[c, pl.ds(off, KS2)],
                                         buf, sem)

        pltpu.sync_copy(d_idx.at[c, s], idx2)
        rdesc(0, m0, r0s).start()
        rdesc(1, m1, r1s).start()
        pltpu.sync_copy(z_hbm.at[pl.ds(row0, NSR)],
                        agg_sh.at[pl.ds(row0, NSR)])

        @pl.when(s == 15)
        def _zero_tail():
            pltpu.sync_copy(z_hbm.at[pl.ds(16 * NSR, NTAIL)],
                            agg_sh.at[pl.ds(16 * NSR, NTAIL)])

        plsc.subcore_barrier()

        def body(g, carry):
            i0 = 2 * g
            rdesc(i0, m0, r0s).wait()
            pltpu.sync_copy(m0, agg_sh.at[idx2.at[i0]], add=True)
            rdesc(i0 + 2, m0, r0s).start()
            rdesc(i0 + 1, m1, r1s).wait()
            pltpu.sync_copy(m1, agg_sh.at[idx2.at[i0 + 1]], add=True)

            @pl.when(g < NPAIR - 1)
            def _refill():
                rdesc(i0 + 3, m1, r1s).start()

            return carry

        lax.fori_loop(0, NPAIR, body, 0)
        if NCS % 2 == 1:
            rdesc(NCS - 1, m0, r0s).wait()
            pltpu.sync_copy(m0, agg_sh.at[idx2.at[NCS - 1]], add=True)
        else:
            rdesc(NCS - 2, m0, r0s).wait()
            pltpu.sync_copy(m0, agg_sh.at[idx2.at[NCS - 2]], add=True)
            rdesc(NCS - 1, m1, r1s).start()
            rdesc(NCS - 1, m1, r1s).wait()
            pltpu.sync_copy(m1, agg_sh.at[idx2.at[NCS - 1]], add=True)
        plsc.subcore_barrier()
        pltpu.sync_copy(agg_sh.at[pl.ds(row0, NSR)],
                        agg_out.at[c, pl.ds(row0, NSR)])

        @pl.when(s == 15)
        def _write_tail():
            pltpu.sync_copy(agg_sh.at[pl.ds(16 * NSR, NTAIL)],
                            agg_out.at[c, pl.ds(16 * NSR, NTAIL)])

    return k(m_all, dst2, zeros_nh)


def _tc_init(cq2, emb, w1s, w1d, b1):
    def body(q_ref, e_ref, ws_ref, wd_ref, b1_ref, x_ref, a_ref, bt_ref):
        q = q_ref[...]
        e = e_ref[...]
        x = jnp.where(q == 0, e[0:1, :], jnp.where(q == 1, e[1:2, :], e[2:3, :]))
        x_ref[...] = x
        for t in range(2):
            a_ref[t] = jnp.dot(x, ws_ref[t], preferred_element_type=jnp.float32)
            bt_ref[t] = (jnp.dot(x, wd_ref[t],
                                 preferred_element_type=jnp.float32)
                         + b1_ref[t])

    return pl.pallas_call(
        body,
        grid=(N // BN,),
        in_specs=[
            pl.BlockSpec((BN, 1), lambda i: (i, 0)),
            pl.BlockSpec((3, H), lambda i: (0, 0)),
            pl.BlockSpec((2, H, H), lambda i: (0, 0, 0)),
            pl.BlockSpec((2, H, H), lambda i: (0, 0, 0)),
            pl.BlockSpec((2, 1, H), lambda i: (0, 0, 0)),
        ],
        out_specs=[
            pl.BlockSpec((BN, H), lambda i: (i, 0)),
            pl.BlockSpec((2, BN, H), lambda i: (0, i, 0)),
            pl.BlockSpec((2, BN, H), lambda i: (0, i, 0)),
        ],
        out_shape=[
            jax.ShapeDtypeStruct((N, H), jnp.float32),
            jax.ShapeDtypeStruct((2, N, H), jnp.float32),
            jax.ShapeDtypeStruct((2, N, H), jnp.float32),
        ],
    )(cq2, emb, w1s, w1d, b1)


def _tc_mlp(z_all, w2, b2, w3, b3, w4, b4):
    def body(z_ref, w2r, b2r, w3r, b3r, w4r, b4r, m_ref):
        t = jnp.maximum(z_ref[0], 0.0).astype(jnp.bfloat16)
        t = jnp.maximum(
            jnp.dot(t, w2r[0], preferred_element_type=jnp.float32) + b2r[0],
            0.0).astype(jnp.bfloat16)
        t = jnp.maximum(
            jnp.dot(t, w3r[0], preferred_element_type=jnp.float32) + b3r[0],
            0.0).astype(jnp.bfloat16)
        m_ref[0] = jnp.dot(t, w4r[0], preferred_element_type=jnp.float32) + b4r[0]

    ne = z_all.shape[1]
    wspec = pl.BlockSpec((1, H, H), lambda t, i: (t, 0, 0))
    bspec = pl.BlockSpec((1, 1, H), lambda t, i: (t, 0, 0))
    return pl.pallas_call(
        body,
        grid=(2, ne // BE),
        in_specs=[pl.BlockSpec((1, BE, H), lambda t, i: (t, i, 0)),
                  wspec, bspec, wspec, bspec, wspec, bspec],
        out_specs=pl.BlockSpec((1, BE, H), lambda t, i: (t, i, 0)),
        out_shape=jax.ShapeDtypeStruct((2, ne, H), jnp.float32),
    )(z_all, w2, b2, w3, b3, w4, b4)


def _tc_lstm(x, agg0, agg1, rh, rc, wih, whh, w1s, w1d, b1, wsc):
    def body(x_ref, g0_ref, g1_ref, h_ref, c_ref, wih_ref, whh_ref, ws_ref,
             wd_ref, b1_ref, sc_ref, h2_ref, c2_ref, a_ref, bt_ref, lg_ref):
        xb = x_ref[...]
        gates = (
            jnp.dot(xb, wih_ref[0:H], preferred_element_type=jnp.float32)
            + jnp.dot(g0_ref[0] + g1_ref[0], wih_ref[H:2 * H],
                      preferred_element_type=jnp.float32)
            + jnp.dot(g0_ref[1] + g1_ref[1], wih_ref[2 * H:3 * H],
                      preferred_element_type=jnp.float32)
            + jnp.dot(h_ref[...], whh_ref[...],
                      preferred_element_type=jnp.float32))
        i_g = gates[:, 0:H]
        f_g = gates[:, H:2 * H]
        g_g = gates[:, 2 * H:3 * H]
        o_g = gates[:, 3 * H:4 * H]
        c_new = (jax.nn.sigmoid(f_g) * c_ref[...]
                 + jax.nn.sigmoid(i_g) * jnp.tanh(g_g))
        h_new = jax.nn.sigmoid(o_g) * jnp.tanh(c_new)
        c2_ref[...] = c_new
        h2_ref[...] = h_new
        for t in range(2):
            a_ref[t] = jnp.dot(h_new, ws_ref[t],
                               preferred_element_type=jnp.float32)
            bt_ref[t] = (jnp.dot(h_new, wd_ref[t],
                                 preferred_element_type=jnp.float32)
                         + b1_ref[t])
        lg_ref[...] = jnp.sum(h_new * sc_ref[...], axis=1, keepdims=True)

    return pl.pallas_call(
        body,
        grid=(N // BN,),
        in_specs=[
            pl.BlockSpec((BN, H), lambda i: (i, 0)),
            pl.BlockSpec((2, BN, H), lambda i: (0, i, 0)),
            pl.BlockSpec((2, BN, H), lambda i: (0, i, 0)),
            pl.BlockSpec((BN, H), lambda i: (i, 0)),
            pl.BlockSpec((BN, H), lambda i: (i, 0)),
            pl.BlockSpec((3 * H, 4 * H), lambda i: (0, 0)),
            pl.BlockSpec((H, 4 * H), lambda i: (0, 0)),
            pl.BlockSpec((2, H, H), lambda i: (0, 0, 0)),
            pl.BlockSpec((2, H, H), lambda i: (0, 0, 0)),
            pl.BlockSpec((2, 1, H), lambda i: (0, 0, 0)),
            pl.BlockSpec((1, H), lambda i: (0, 0)),
        ],
        out_specs=[
            pl.BlockSpec((BN, H), lambda i: (i, 0)),
            pl.BlockSpec((BN, H), lambda i: (i, 0)),
            pl.BlockSpec((2, BN, H), lambda i: (0, i, 0)),
            pl.BlockSpec((2, BN, H), lambda i: (0, i, 0)),
            pl.BlockSpec((BN, 1), lambda i: (i, 0)),
        ],
        out_shape=[
            jax.ShapeDtypeStruct((N, H), jnp.float32),
            jax.ShapeDtypeStruct((N, H), jnp.float32),
            jax.ShapeDtypeStruct((2, N, H), jnp.float32),
            jax.ShapeDtypeStruct((2, N, H), jnp.float32),
            jax.ShapeDtypeStruct((N, 1), jnp.float32),
        ],
    )(x, agg0, agg1, rh, rc, wih, whh, w1s, w1d, b1, wsc)


def kernel(cell_q, edge_intra, edge_inter, params):
    p = params
    cq2 = cell_q.astype(jnp.int32).reshape(N, 1)
    ei = edge_intra.astype(jnp.int32)
    ee = edge_inter.astype(jnp.int32)
    EHT = [96000, 64000]
    NCHS = [EHT[0] * 2 // NW // KG, EHT[1] * 2 // NW // KG]

    def _half(a_i, a_e, h, off):
        sl = slice(0, EHT[0]) if h == 0 else slice(EHT[0], E)
        return jnp.concatenate([a_i[sl], a_e[sl] + off])

    src_h = [_half(ei[0], ee[0], h, N).reshape(NW, NCHS[h], KG)
             for h in (0, 1)]
    dst_gh = [_half(ei[1], ee[1], h, N).reshape(NW, NCHS[h], KG)
              for h in (0, 1)]
    dst_sh = [_half(ei[1], ee[1], h, 0).reshape(2, 16, NCHS[h], KG)
              for h in (0, 1)]

    w1s = jnp.stack([p['intra_Ws'][0][:H], p['inter_Ws'][0][:H]])
    w1d = jnp.stack([p['intra_Ws'][0][H:], p['inter_Ws'][0][H:]])
    b1 = jnp.stack([p['intra_bs'][0], p['inter_bs'][0]])[:, None, :]
    w2 = jnp.stack([p['intra_Ws'][1], p['inter_Ws'][1]]).astype(jnp.bfloat16)
    b2 = jnp.stack([p['intra_bs'][1], p['inter_bs'][1]])[:, None, :]
    w3 = jnp.stack([p['intra_Ws'][2], p['inter_Ws'][2]]).astype(jnp.bfloat16)
    b3 = jnp.stack([p['intra_bs'][2], p['inter_bs'][2]])[:, None, :]
    w4 = jnp.stack([p['intra_Ws'][3], p['inter_Ws'][3]]).astype(jnp.bfloat16)
    b4 = jnp.stack([p['intra_bs'][3], p['inter_bs'][3]])[:, None, :]
    zeros_nh = jnp.zeros((N, H), jnp.float32)

    x, A, B = _tc_init(cq2, p['digit_embed'], w1s, w1d, b1)
    rh = zeros_nh
    rc = zeros_nh
    lg = None
    for _ in range(STEPS):
        A2 = A.reshape(2 * N, H)
        B2 = B.reshape(2 * N, H)
        z0 = _sc_gather_add(A2, B2, src_h[0], dst_gh[0]).reshape(2, EHT[0], H)
        z1 = _sc_gather_add(A2, B2, src_h[1], dst_gh[1]).reshape(2, EHT[1], H)
        m0 = _tc_mlp(z0, w2, b2, w3, b3, w4, b4)
        m1 = _tc_mlp(z1, w2, b2, w3, b3, w4, b4)
        ag0 = _sc_scatter_add(m0, dst_sh[0], zeros_nh)
        ag1 = _sc_scatter_add(m1, dst_sh[1], zeros_nh)
        rh, rc, A, B, lg = _tc_lstm(x, ag0, ag1, rh, rc, p['W_ih'],
                                    p['W_hh'], w1s, w1d, b1,
                                    p['w_score'][None, :])
    return lg[:, 0]
